# Initial kernel scaffold; baseline (speedup 1.0000x reference)
import jax
import jax.numpy as jnp
from jax.experimental import pallas as pl


def kernel(embed_id, weight):
    raise NotImplementedError("write your pallas kernel here")


# speedup vs baseline: 2.4087x; 2.4087x over previous
import functools

import jax
import jax.numpy as jnp
from jax import lax
from jax.experimental import pallas as pl
from jax.experimental.pallas import tpu as pltpu
from jax.experimental.pallas import tpu_sc as plsc

_NUM_CORES = 2
_NUM_SUBCORES = 16
_NW = _NUM_CORES * _NUM_SUBCORES
_CHUNK = 128
_NBUF = 3


@functools.lru_cache(maxsize=None)
def _make_gather(b: int, d: int):
    assert b % (_NW * _CHUNK) == 0
    b_per_w = b // _NW
    n_chunks = b_per_w // _CHUNK

    mesh = plsc.VectorSubcoreMesh(core_axis_name="c", subcore_axis_name="s")
    scratch = [pltpu.VMEM((n_chunks, _CHUNK), jnp.int32)]
    scratch += [pltpu.VMEM((_CHUNK, d), jnp.float32) for _ in range(_NBUF)]
    scratch += [pltpu.SemaphoreType.DMA for _ in range(2 * _NBUF)]

    @functools.partial(
        pl.kernel,
        mesh=mesh,
        out_type=jax.ShapeDtypeStruct((b, d), jnp.float32),
        scratch_types=scratch,
    )
    def gather_kernel(idx_hbm, table_hbm, out_hbm, idx_v, *rest):
        bufs = rest[:_NBUF]
        gsems = rest[_NBUF:2 * _NBUF]
        ssems = rest[2 * _NBUF:]
        wid = lax.axis_index("s") * _NUM_CORES + lax.axis_index("c")
        base = wid * b_per_w
        pltpu.sync_copy(idx_hbm.at[wid], idx_v)

        def start_gather(c):
            return pltpu.async_copy(
                table_hbm.at[idx_v.at[c]], bufs[c % _NBUF], gsems[c % _NBUF])

        def start_scatter(c):
            return pltpu.async_copy(
                bufs[c % _NBUF],
                out_hbm.at[pl.ds(base + c * _CHUNK, _CHUNK)],
                ssems[c % _NBUF])

        gath = {}
        scat = {}
        for c in range(min(_NBUF, n_chunks)):
            gath[c] = start_gather(c)
        for c in range(n_chunks):
            gath[c].wait()
            scat[c] = start_scatter(c)
            nxt = c + _NBUF
            if nxt < n_chunks:
                scat.pop(c).wait()
                gath[nxt] = start_gather(nxt)
        for c in sorted(scat):
            scat[c].wait()

    return gather_kernel


def kernel(embed_id, weight):
    orig_shape = embed_id.shape
    d = weight.shape[1]
    b = embed_id.size
    idx = jnp.asarray(embed_id, jnp.int32).reshape(
        _NW, b // (_NW * _CHUNK), _CHUNK)
    out = _make_gather(b, d)(idx, weight)
    return out.reshape(*orig_shape, d)


# speedup vs baseline: 2.4174x; 1.0036x over previous
import functools

import jax
import jax.numpy as jnp
from jax import lax
from jax.experimental import pallas as pl
from jax.experimental.pallas import tpu as pltpu
from jax.experimental.pallas import tpu_sc as plsc

_NUM_CORES = 2
_NUM_SUBCORES = 16
_NW = _NUM_CORES * _NUM_SUBCORES
_CHUNK = 64
_NBUF = 7


@functools.lru_cache(maxsize=None)
def _make_gather(b0: int, b1: int, d: int):
    b = b0 * b1
    assert b % (_NW * _CHUNK) == 0 and b1 % _NW == 0 or _NW % b1 == 0
    b_per_w = b // _NW
    n_chunks = b_per_w // _CHUNK
    w_per_row = _NW // b0

    mesh = plsc.VectorSubcoreMesh(core_axis_name="c", subcore_axis_name="s")
    scratch = [pltpu.VMEM((b_per_w,), jnp.int32)]
    scratch += [pltpu.VMEM((_CHUNK, d), jnp.float32) for _ in range(_NBUF)]
    scratch += [pltpu.SemaphoreType.DMA for _ in range(2 * _NBUF)]

    @functools.partial(
        pl.kernel,
        mesh=mesh,
        out_type=jax.ShapeDtypeStruct((b, d), jnp.float32),
        scratch_types=scratch,
    )
    def gather_kernel(idx_hbm, table_hbm, out_hbm, idx_v, *rest):
        bufs = rest[:_NBUF]
        gsems = rest[_NBUF:2 * _NBUF]
        ssems = rest[2 * _NBUF:]
        wid = lax.axis_index("s") * _NUM_CORES + lax.axis_index("c")
        base = wid * b_per_w
        row = wid // w_per_row
        col = (wid % w_per_row) * b_per_w
        pltpu.sync_copy(idx_hbm.at[row, pl.ds(col, b_per_w)], idx_v)

        def start_gather(c):
            return pltpu.async_copy(
                table_hbm.at[idx_v.at[pl.ds(c * _CHUNK, _CHUNK)]],
                bufs[c % _NBUF], gsems[c % _NBUF])

        def start_scatter(c):
            return pltpu.async_copy(
                bufs[c % _NBUF],
                out_hbm.at[pl.ds(base + c * _CHUNK, _CHUNK)],
                ssems[c % _NBUF])

        gath = {}
        scat = {}
        for c in range(min(_NBUF, n_chunks)):
            gath[c] = start_gather(c)
        for c in range(n_chunks):
            gath[c].wait()
            scat[c] = start_scatter(c)
            nxt = c + _NBUF
            if nxt < n_chunks:
                scat.pop(c).wait()
                gath[nxt] = start_gather(nxt)
        for c in sorted(scat):
            scat[c].wait()

    return gather_kernel


def kernel(embed_id, weight):
    b0, b1 = embed_id.shape
    d = weight.shape[1]
    idx = jnp.asarray(embed_id, jnp.int32)
    out = _make_gather(b0, b1, d)(idx, weight)
    return out.reshape(b0, b1, d)
